# baseline (speedup 1.0000x reference)
import functools

import jax
import jax.numpy as jnp
from jax import lax
from jax.experimental import pallas as pl
from jax.experimental.pallas import tpu as pltpu

_GI, _GJ = 64, 128
_N = _GI * _GJ
_D = 32
_NL = 10
_BM = 256


def _som_block(x_ref, wtn_ref, lp_ref, out_ref):
    xb = x_ref[...]
    wtn = wtn_ref[...]
    w2 = 0.25 * jnp.sum(wtn * wtn, axis=0, keepdims=True)
    k = jnp.dot(xb, wtn, preferred_element_type=jnp.float32) + w2

    m = jnp.min(k, axis=1, keepdims=True)
    idx = lax.broadcasted_iota(jnp.int32, k.shape, 1)
    bmu = jnp.min(jnp.where(k <= m, idx, _N), axis=1, keepdims=True)

    pi = (bmu // _GJ).astype(jnp.float32)
    pj = (bmu % _GJ).astype(jnp.float32)

    gi = lax.broadcasted_iota(jnp.int32, (xb.shape[0], _GI), 1).astype(jnp.float32)
    gj = lax.broadcasted_iota(jnp.int32, (xb.shape[0], _GJ), 1).astype(jnp.float32)
    ai = jnp.exp(-0.5 * (pi - gi) ** 2)
    cj = jnp.exp(-0.5 * (pj - gj) ** 2)
    norm = jnp.sum(ai, axis=1, keepdims=True) * jnp.sum(cj, axis=1, keepdims=True)

    mm = jnp.dot(cj, lp_ref[...], preferred_element_type=jnp.float32)

    col = lax.broadcasted_iota(jnp.int32, (_GI, _GI * _NL), 1)
    row = lax.broadcasted_iota(jnp.int32, (_GI, _GI * _NL), 0)
    rmat = (col // _NL == row).astype(jnp.float32)
    scol = lax.broadcasted_iota(jnp.int32, (_GI * _NL, _NL), 0)
    srow = lax.broadcasted_iota(jnp.int32, (_GI * _NL, _NL), 1)
    smat = (scol % _NL == srow).astype(jnp.float32)

    ae = jnp.dot(ai, rmat, preferred_element_type=jnp.float32)
    out = jnp.dot(ae * mm, smat, preferred_element_type=jnp.float32)
    out_ref[...] = out / norm


@jax.jit
def kernel(x, W, L):
    B = x.shape[0]
    x = x.reshape(B, -1)
    wt = (-2.0 * W).T
    lp = L.reshape(_GI, _GJ, _NL).transpose(1, 0, 2).reshape(_GJ, _GI * _NL)
    grid = (B // _BM,)
    return pl.pallas_call(
        _som_block,
        grid=grid,
        in_specs=[
            pl.BlockSpec((_BM, _D), lambda i: (i, 0)),
            pl.BlockSpec((_D, _N), lambda i: (0, 0)),
            pl.BlockSpec((_GJ, _GI * _NL), lambda i: (0, 0)),
        ],
        out_specs=pl.BlockSpec((_BM, _NL), lambda i: (i, 0)),
        out_shape=jax.ShapeDtypeStruct((B, _NL), jnp.float32),
        compiler_params=pltpu.CompilerParams(
            dimension_semantics=("parallel",),
        ),
    )(x, wt, lp)


# speedup vs baseline: 1.3242x; 1.3242x over previous
import functools

import jax
import jax.numpy as jnp
from jax import lax
from jax.experimental import pallas as pl
from jax.experimental.pallas import tpu as pltpu

_GI, _GJ = 64, 128
_N = _GI * _GJ
_D = 32
_NL = 10
_BM = 256
_RG = 128
_NC = 1024
_BIG = 3.0e38


def _argmin_rows(xg, wtn, w2):
    rg = xg.shape[0]
    m = jnp.full((rg, _GJ), _BIG, dtype=jnp.float32)
    c = jnp.zeros((rg, _GJ), dtype=jnp.int32)
    for ch in range(_N // _NC):
        kc = jnp.dot(xg, wtn[:, ch * _NC:(ch + 1) * _NC],
                     preferred_element_type=jnp.float32)
        for j in range(_NC // _GJ):
            jj = ch * (_NC // _GJ) + j
            v = kc[:, j * _GJ:(j + 1) * _GJ] + w2[:, jj * _GJ:(jj + 1) * _GJ]
            upd = v < m
            c = jnp.where(upd, jj, c)
            m = jnp.minimum(m, v)
    rowmin = jnp.min(m, axis=1, keepdims=True)
    lane = lax.broadcasted_iota(jnp.int32, (rg, _GJ), 1)
    flat = c * _GJ + lane
    cand = jnp.where(m <= rowmin, flat, _N)
    return jnp.min(cand, axis=1, keepdims=True)


def _som_block(x_ref, wtn_ref, lp_ref, out_ref):
    xb = x_ref[...]
    wtn = wtn_ref[...]
    w2 = 0.25 * jnp.sum(wtn * wtn, axis=0, keepdims=True)

    bmu = jnp.concatenate(
        [_argmin_rows(xb[r * _RG:(r + 1) * _RG], wtn, w2)
         for r in range(_BM // _RG)], axis=0)

    pi = (bmu // _GJ).astype(jnp.float32)
    pj = (bmu % _GJ).astype(jnp.float32)

    gi = lax.broadcasted_iota(jnp.int32, (_BM, _GI), 1).astype(jnp.float32)
    gj = lax.broadcasted_iota(jnp.int32, (_BM, _GJ), 1).astype(jnp.float32)
    ai = jnp.exp(-0.5 * (pi - gi) ** 2)
    cj = jnp.exp(-0.5 * (pj - gj) ** 2)
    norm = jnp.sum(ai, axis=1, keepdims=True) * jnp.sum(cj, axis=1, keepdims=True)

    mm = jnp.dot(cj, lp_ref[...], preferred_element_type=jnp.float32)

    col = lax.broadcasted_iota(jnp.int32, (_GI, _GI * _NL), 1)
    row = lax.broadcasted_iota(jnp.int32, (_GI, _GI * _NL), 0)
    rmat = (col // _NL == row).astype(jnp.float32)
    scol = lax.broadcasted_iota(jnp.int32, (_GI * _NL, _NL), 0)
    srow = lax.broadcasted_iota(jnp.int32, (_GI * _NL, _NL), 1)
    smat = (scol % _NL == srow).astype(jnp.float32)

    ae = jnp.dot(ai, rmat, preferred_element_type=jnp.float32)
    out = jnp.dot(ae * mm, smat, preferred_element_type=jnp.float32)
    out_ref[...] = out / norm


@jax.jit
def kernel(x, W, L):
    B = x.shape[0]
    x = x.reshape(B, -1)
    wt = (-2.0 * W).T
    lp = L.reshape(_GI, _GJ, _NL).transpose(1, 0, 2).reshape(_GJ, _GI * _NL)
    grid = (B // _BM,)
    return pl.pallas_call(
        _som_block,
        grid=grid,
        in_specs=[
            pl.BlockSpec((_BM, _D), lambda i: (i, 0)),
            pl.BlockSpec((_D, _N), lambda i: (0, 0)),
            pl.BlockSpec((_GJ, _GI * _NL), lambda i: (0, 0)),
        ],
        out_specs=pl.BlockSpec((_BM, _NL), lambda i: (i, 0)),
        out_shape=jax.ShapeDtypeStruct((B, _NL), jnp.float32),
        compiler_params=pltpu.CompilerParams(
            dimension_semantics=("parallel",),
        ),
    )(x, wt, lp)


# speedup vs baseline: 1.5506x; 1.1709x over previous
import functools

import jax
import jax.numpy as jnp
from jax import lax
from jax.experimental import pallas as pl
from jax.experimental.pallas import tpu as pltpu

_GI, _GJ = 64, 128
_N = _GI * _GJ
_D = 32
_NL = 10
_BM = 512
_RG = 128
_NC = 1024
_BIG = 3.0e38


def _argmin_rows(xg, wtn, w2):
    rg = xg.shape[0]
    m = jnp.full((rg, _GJ), _BIG, dtype=jnp.float32)
    c = jnp.zeros((rg, _GJ), dtype=jnp.int32)
    for ch in range(_N // _NC):
        kc = jnp.dot(xg, wtn[:, ch * _NC:(ch + 1) * _NC],
                     preferred_element_type=jnp.float32)
        for j in range(_NC // _GJ):
            jj = ch * (_NC // _GJ) + j
            v = kc[:, j * _GJ:(j + 1) * _GJ] + w2[:, jj * _GJ:(jj + 1) * _GJ]
            upd = v < m
            c = jnp.where(upd, jj, c)
            m = jnp.minimum(m, v)
    rowmin = jnp.min(m, axis=1, keepdims=True)
    lane = lax.broadcasted_iota(jnp.int32, (rg, _GJ), 1)
    flat = c * _GJ + lane
    cand = jnp.where(m <= rowmin, flat, _N)
    return jnp.min(cand, axis=1, keepdims=True)


def _som_block(x_ref, wtn_ref, lp_ref, out_ref, w2_ref):
    xb = x_ref[...]
    wtn = wtn_ref[...]

    @pl.when(pl.program_id(0) == 0)
    def _():
        w2_ref[...] = 0.25 * jnp.sum(wtn * wtn, axis=0, keepdims=True)

    w2 = w2_ref[...]

    bmu = jnp.concatenate(
        [_argmin_rows(xb[r * _RG:(r + 1) * _RG], wtn, w2)
         for r in range(_BM // _RG)], axis=0)

    pi = (bmu // _GJ).astype(jnp.float32)
    pj = (bmu % _GJ).astype(jnp.float32)

    gi = lax.broadcasted_iota(jnp.int32, (_BM, _GI), 1).astype(jnp.float32)
    gj = lax.broadcasted_iota(jnp.int32, (_BM, _GJ), 1).astype(jnp.float32)
    ai = jnp.exp(-0.5 * (pi - gi) ** 2)
    cj = jnp.exp(-0.5 * (pj - gj) ** 2)
    norm = jnp.sum(ai, axis=1, keepdims=True) * jnp.sum(cj, axis=1, keepdims=True)

    mm = jnp.dot(cj, lp_ref[...], preferred_element_type=jnp.float32)

    col = lax.broadcasted_iota(jnp.int32, (_GI, _GI * _NL), 1)
    row = lax.broadcasted_iota(jnp.int32, (_GI, _GI * _NL), 0)
    rmat = (col // _NL == row).astype(jnp.float32)
    scol = lax.broadcasted_iota(jnp.int32, (_GI * _NL, _NL), 0)
    srow = lax.broadcasted_iota(jnp.int32, (_GI * _NL, _NL), 1)
    smat = (scol % _NL == srow).astype(jnp.float32)

    ae = jnp.dot(ai, rmat, preferred_element_type=jnp.float32)
    out = jnp.dot(ae * mm, smat, preferred_element_type=jnp.float32)
    out_ref[...] = out / norm


@jax.jit
def kernel(x, W, L):
    B = x.shape[0]
    x = x.reshape(B, -1)
    wt = (-2.0 * W).T
    lp = L.reshape(_GI, _GJ, _NL).transpose(1, 0, 2).reshape(_GJ, _GI * _NL)
    grid = (B // _BM,)
    return pl.pallas_call(
        _som_block,
        grid=grid,
        in_specs=[
            pl.BlockSpec((_BM, _D), lambda i: (i, 0)),
            pl.BlockSpec((_D, _N), lambda i: (0, 0)),
            pl.BlockSpec((_GJ, _GI * _NL), lambda i: (0, 0)),
        ],
        out_specs=pl.BlockSpec((_BM, _NL), lambda i: (i, 0)),
        out_shape=jax.ShapeDtypeStruct((B, _NL), jnp.float32),
        scratch_shapes=[pltpu.VMEM((1, _N), jnp.float32)],
        compiler_params=pltpu.CompilerParams(
            dimension_semantics=("arbitrary",),
        ),
    )(x, wt, lp)


# speedup vs baseline: 1.6609x; 1.0711x over previous
import functools

import jax
import jax.numpy as jnp
from jax import lax
from jax.experimental import pallas as pl
from jax.experimental.pallas import tpu as pltpu

_GI, _GJ = 64, 128
_N = _GI * _GJ
_D = 32
_NL = 10
_BM = 1024
_RG = 128
_NC = 1024
_BIG = 3.0e38


def _argmin_rows(xg, wtn, w2):
    rg = xg.shape[0]
    m = jnp.full((rg, _GJ), _BIG, dtype=jnp.float32)
    c = jnp.zeros((rg, _GJ), dtype=jnp.int32)
    for ch in range(_N // _NC):
        kc = jnp.dot(xg, wtn[:, ch * _NC:(ch + 1) * _NC],
                     preferred_element_type=jnp.float32)
        for j in range(_NC // _GJ):
            jj = ch * (_NC // _GJ) + j
            v = kc[:, j * _GJ:(j + 1) * _GJ] + w2[:, jj * _GJ:(jj + 1) * _GJ]
            upd = v < m
            c = jnp.where(upd, jj, c)
            m = jnp.minimum(m, v)
    rowmin = jnp.min(m, axis=1, keepdims=True)
    lane = lax.broadcasted_iota(jnp.int32, (rg, _GJ), 1)
    flat = c * _GJ + lane
    cand = jnp.where(m <= rowmin, flat, _N)
    return jnp.min(cand, axis=1, keepdims=True)


def _som_block(x_ref, wtn_ref, lp_ref, out_ref, w2_ref):
    xb = x_ref[...]
    wtn = wtn_ref[...]

    @pl.when(pl.program_id(0) == 0)
    def _():
        w2_ref[...] = 0.25 * jnp.sum(wtn * wtn, axis=0, keepdims=True)

    w2 = w2_ref[...]

    bmu = jnp.concatenate(
        [_argmin_rows(xb[r * _RG:(r + 1) * _RG], wtn, w2)
         for r in range(_BM // _RG)], axis=0)

    pi = (bmu // _GJ).astype(jnp.float32)
    pj = (bmu % _GJ).astype(jnp.float32)

    gi = lax.broadcasted_iota(jnp.int32, (_BM, _GI), 1).astype(jnp.float32)
    gj = lax.broadcasted_iota(jnp.int32, (_BM, _GJ), 1).astype(jnp.float32)
    ai = jnp.exp(-0.5 * (pi - gi) ** 2)
    cj = jnp.exp(-0.5 * (pj - gj) ** 2)
    norm = jnp.sum(ai, axis=1, keepdims=True) * jnp.sum(cj, axis=1, keepdims=True)

    mm = jnp.dot(cj, lp_ref[...], preferred_element_type=jnp.float32)

    col = lax.broadcasted_iota(jnp.int32, (_GI, _GI * _NL), 1)
    row = lax.broadcasted_iota(jnp.int32, (_GI, _GI * _NL), 0)
    rmat = (col // _NL == row).astype(jnp.float32)
    scol = lax.broadcasted_iota(jnp.int32, (_GI * _NL, _NL), 0)
    srow = lax.broadcasted_iota(jnp.int32, (_GI * _NL, _NL), 1)
    smat = (scol % _NL == srow).astype(jnp.float32)

    ae = jnp.dot(ai, rmat, preferred_element_type=jnp.float32)
    out = jnp.dot(ae * mm, smat, preferred_element_type=jnp.float32)
    out_ref[...] = out / norm


@jax.jit
def kernel(x, W, L):
    B = x.shape[0]
    x = x.reshape(B, -1)
    wt = (-2.0 * W).T
    lp = L.reshape(_GI, _GJ, _NL).transpose(1, 0, 2).reshape(_GJ, _GI * _NL)
    grid = (B // _BM,)
    return pl.pallas_call(
        _som_block,
        grid=grid,
        in_specs=[
            pl.BlockSpec((_BM, _D), lambda i: (i, 0)),
            pl.BlockSpec((_D, _N), lambda i: (0, 0)),
            pl.BlockSpec((_GJ, _GI * _NL), lambda i: (0, 0)),
        ],
        out_specs=pl.BlockSpec((_BM, _NL), lambda i: (i, 0)),
        out_shape=jax.ShapeDtypeStruct((B, _NL), jnp.float32),
        scratch_shapes=[pltpu.VMEM((1, _N), jnp.float32)],
        compiler_params=pltpu.CompilerParams(
            dimension_semantics=("arbitrary",),
        ),
    )(x, wt, lp)
